# baseline (speedup 1.0000x reference)
import jax
import jax.numpy as jnp
from jax import lax
from jax.experimental import pallas as pl
from jax.experimental.pallas import tpu as pltpu
from jax.experimental.pallas import tpu_sc as plsc

N = 10000
E = 320000
D = 128
DP = 16
NG = 128
NC, NS = 2, 16
NW = NC * NS
EPT = E // NW
SC16 = 16
NSUB = EPT // SC16
C = 80
NCH = EPT // C
SUB = C // SC16
NRING = 3
PH0N = 63
PHASES = ((0, 63), (63, 62))
NP = 10112
RPT = NP // NS
RB = 1000
GRID = N // RB


def _make_edge_pass(feat, gather):
    mesh = plsc.VectorSubcoreMesh(core_axis_name="c", subcore_axis_name="s")
    scratch = [
        pltpu.VMEM((NCH, C), jnp.int32),
        pltpu.VMEM_SHARED((NP, feat), jnp.float32),
        pltpu.SemaphoreType.DMA,
    ]
    if gather:
        scratch.insert(0, pltpu.VMEM((PH0N, C), jnp.int32))
        for _ in range(NRING):
            scratch.append(pltpu.VMEM((C, feat), jnp.float32))
        for _ in range(NRING):
            scratch.append(pltpu.SemaphoreType.DMA)
    else:
        scratch.append(pltpu.VMEM((SC16, feat), jnp.float32))

    if gather:
        def body(hp, srcr, dstr, zer, out, isrc, idst, acc, ssem, *bufs):
            rows = bufs[:NRING]
            gsem = bufs[NRING:]
            c = lax.axis_index("c")
            s = lax.axis_index("s")
            wid = c * NS + s
            rbase = s * RPT
            pltpu.sync_copy(zer.at[pl.ds(rbase, RPT)],
                            acc.at[pl.ds(rbase, RPT)])
            pltpu.sync_copy(dstr.at[wid], idst)
            plsc.subcore_barrier()

            def gath(j, b):
                return pltpu.make_async_copy(hp.at[isrc.at[j]],
                                             rows[b], gsem[b])

            def scat(j, buf):
                for u in range(SUB):
                    pltpu.async_copy(
                        buf.at[pl.ds(u * SC16, SC16)],
                        acc.at[idst.at[j, pl.ds(u * SC16, SC16)]],
                        ssem, add=True)
                for u in range(SUB):
                    pltpu.make_async_copy(
                        buf.at[pl.ds(u * SC16, SC16)],
                        acc.at[idst.at[j, pl.ds(u * SC16, SC16)]],
                        ssem).wait()

            for ph0, phn in PHASES:
                pltpu.sync_copy(srcr.at[wid, pl.ds(ph0, phn)],
                                isrc.at[pl.ds(0, phn)])

                for b in range(NRING):
                    gath(b, b).start()

                def ring(k, carry):
                    for b in range(NRING):
                        jl = k * NRING + b
                        gath(jl, b).wait()
                        scat(ph0 + jl, rows[b])

                        @pl.when(jl + NRING < phn)
                        def _():
                            gath(jl + NRING, b).start()
                    return carry

                lax.fori_loop(0, phn // NRING, ring, 0)
                for r in range(phn % NRING):
                    jl = phn - (phn % NRING) + r
                    gath(jl, jl % NRING).wait()
                    scat(ph0 + jl, rows[jl % NRING])
            plsc.subcore_barrier()
            pltpu.sync_copy(acc.at[pl.ds(rbase, RPT)],
                            out.at[c, pl.ds(rbase, RPT)])
    else:
        def body(ones, dstr, zer, out, idst, acc, ssem, rows0):
            c = lax.axis_index("c")
            s = lax.axis_index("s")
            wid = c * NS + s
            rbase = s * RPT
            pltpu.sync_copy(zer.at[pl.ds(rbase, RPT)],
                            acc.at[pl.ds(rbase, RPT)])
            pltpu.sync_copy(dstr.at[wid], idst)
            pltpu.sync_copy(ones, rows0)
            plsc.subcore_barrier()

            def chunk(j, carry):
                for u in range(SUB):
                    pltpu.async_copy(
                        rows0, acc.at[idst.at[j, pl.ds(u * SC16, SC16)]],
                        ssem, add=True)
                for u in range(SUB):
                    pltpu.make_async_copy(
                        rows0, acc.at[idst.at[j, pl.ds(u * SC16, SC16)]],
                        ssem).wait()
                return carry

            lax.fori_loop(0, NCH, chunk, 0)
            plsc.subcore_barrier()
            pltpu.sync_copy(acc.at[pl.ds(rbase, RPT)],
                            out.at[c, pl.ds(rbase, RPT)])

    return pl.kernel(
        body,
        out_type=jax.ShapeDtypeStruct((NC, NP, feat), jnp.float32),
        mesh=mesh,
        scratch_types=scratch,
        compiler_params=pltpu.CompilerParams(use_tc_tiling_on_sc=False),
    )


_edge128 = _make_edge_pass(D, gather=True)
_edge16 = _make_edge_pass(DP, gather=True)
_deg16 = _make_edge_pass(DP, gather=False)


def _mm_body(x_ref, w_ref, h_ref):
    h_ref[...] = jnp.dot(x_ref[...], w_ref[...],
                         preferred_element_type=jnp.float32)


def _matmul_first(x, w):
    return pl.pallas_call(
        _mm_body,
        grid=(GRID,),
        in_specs=[
            pl.BlockSpec((RB, D), lambda i: (i, 0)),
            pl.BlockSpec((D, D), lambda i: (0, 0)),
        ],
        out_specs=pl.BlockSpec((RB, D), lambda i: (i, 0)),
        out_shape=jax.ShapeDtypeStruct((N, D), jnp.float32),
    )(x, w)


def _scale_body(h_ref, deg_ref, hp_ref, dinv_ref):
    deg = deg_ref[0] + deg_ref[1] + 1.0
    dinv = lax.rsqrt(deg)
    dinv_ref[...] = dinv
    hp_ref[...] = h_ref[...] * dinv[:, 0:1]


def _scale_first(h, degp):
    return pl.pallas_call(
        _scale_body,
        grid=(GRID,),
        in_specs=[
            pl.BlockSpec((RB, D), lambda i: (i, 0)),
            pl.BlockSpec((NC, RB, DP), lambda i: (0, i, 0)),
        ],
        out_specs=[
            pl.BlockSpec((RB, D), lambda i: (i, 0)),
            pl.BlockSpec((RB, DP), lambda i: (i, 0)),
        ],
        out_shape=[
            jax.ShapeDtypeStruct((N, D), jnp.float32),
            jax.ShapeDtypeStruct((N, DP), jnp.float32),
        ],
    )(h, degp)


def _k2_body(acc_ref, hprev_ref, dinv_ref, b_ref, w_ref, out_ref):
    dinv = dinv_ref[:, 0:1]
    z = (acc_ref[0] + acc_ref[1] + hprev_ref[...]) * dinv + b_ref[...]
    z = jnp.maximum(z, 0.0)
    out_ref[...] = jnp.dot(z, w_ref[...],
                           preferred_element_type=jnp.float32) * dinv


def _combine_matmul(accp, hprev, dinv16, b2d, w):
    dout = w.shape[1]
    return pl.pallas_call(
        _k2_body,
        grid=(GRID,),
        in_specs=[
            pl.BlockSpec((NC, RB, D), lambda i: (0, i, 0)),
            pl.BlockSpec((RB, D), lambda i: (i, 0)),
            pl.BlockSpec((RB, DP), lambda i: (i, 0)),
            pl.BlockSpec((1, D), lambda i: (0, 0)),
            pl.BlockSpec((D, dout), lambda i: (0, 0)),
        ],
        out_specs=pl.BlockSpec((RB, dout), lambda i: (i, 0)),
        out_shape=jax.ShapeDtypeStruct((N, dout), jnp.float32),
    )(accp, hprev, dinv16, b2d, w)


def _k3_body(acc_ref, h3_ref, dinv_ref, b_ref, batch_ref, out_ref,
             sums_ref, cnt_ref):
    i = pl.program_id(0)

    @pl.when(i == 0)
    def _():
        sums_ref[...] = jnp.zeros_like(sums_ref)
        cnt_ref[...] = jnp.zeros_like(cnt_ref)

    dinv = dinv_ref[:, 0:1]
    z = (acc_ref[0] + acc_ref[1] + h3_ref[...]) * dinv + b_ref[...]
    b = batch_ref[0, 0]
    iota = lax.broadcasted_iota(jnp.int32, (RB, NG), 1)
    onehot = (b[:, None] == iota).astype(jnp.float32)
    sums_ref[...] += lax.dot_general(
        onehot, z, (((0,), (0,)), ((), ())),
        preferred_element_type=jnp.float32)
    cnt_ref[...] += lax.dot_general(
        onehot, jnp.ones((RB, DP), jnp.float32), (((0,), (0,)), ((), ())),
        preferred_element_type=jnp.float32)

    @pl.when(i == GRID - 1)
    def _():
        out_ref[...] = sums_ref[...] / jnp.maximum(cnt_ref[...], 1.0)


def _combine_pool(accp3, h3p, dinv16, b3p2d, batchr):
    return pl.pallas_call(
        _k3_body,
        grid=(GRID,),
        in_specs=[
            pl.BlockSpec((NC, RB, DP), lambda i: (0, i, 0)),
            pl.BlockSpec((RB, DP), lambda i: (i, 0)),
            pl.BlockSpec((RB, DP), lambda i: (i, 0)),
            pl.BlockSpec((1, DP), lambda i: (0, 0)),
            pl.BlockSpec((1, 1, RB), lambda i: (i, 0, 0)),
        ],
        out_specs=pl.BlockSpec((NG, DP), lambda i: (0, 0)),
        out_shape=jax.ShapeDtypeStruct((NG, DP), jnp.float32),
        scratch_shapes=[
            pltpu.VMEM((NG, DP), jnp.float32),
            pltpu.VMEM((NG, DP), jnp.float32),
        ],
        compiler_params=pltpu.CompilerParams(
            dimension_semantics=("arbitrary",)),
    )(accp3, h3p, dinv16, b3p2d, batchr)


def kernel(x, edge_index, batch, W1, b1, W2, b2, W3, b3):
    f32 = jnp.float32
    srcr = edge_index[0].reshape(NW, NCH, C)
    dstr = edge_index[1].reshape(NW, NCH, C)
    zeros128 = jnp.zeros((NP, D), f32)
    zeros16 = jnp.zeros((NP, DP), f32)
    ones16 = jnp.ones((SC16, DP), f32)
    W3p = jnp.pad(W3, ((0, 0), (0, DP - W3.shape[1])))
    b3p = jnp.pad(b3, (0, DP - b3.shape[0])).reshape(1, DP)
    b1_2d = b1.reshape(1, D)
    b2_2d = b2.reshape(1, D)
    batchr = batch.reshape(GRID, 1, RB)

    h1 = _matmul_first(x, W1)
    degp = _deg16(ones16, dstr, zeros16)
    h1p, dinv16 = _scale_first(h1, degp)
    acc1 = _edge128(h1p, srcr, dstr, zeros128)
    h2p = _combine_matmul(acc1, h1p, dinv16, b1_2d, W2)
    acc2 = _edge128(h2p, srcr, dstr, zeros128)
    h3p = _combine_matmul(acc2, h2p, dinv16, b2_2d, W3p)
    acc3 = _edge16(h3p, srcr, dstr, zeros16)
    g16 = _combine_pool(acc3, h3p, dinv16, b3p, batchr)
    return g16[:, :b3.shape[0]]


# speedup vs baseline: 1.0016x; 1.0016x over previous
import jax
import jax.numpy as jnp
from jax import lax
from jax.experimental import pallas as pl
from jax.experimental.pallas import tpu as pltpu
from jax.experimental.pallas import tpu_sc as plsc

N = 10000
E = 320000
D = 128
DP = 16
NG = 128
NC, NS = 2, 16
NW = NC * NS
EPT = E // NW
SC16 = 16
NSUB = EPT // SC16
C = 80
NCH = EPT // C
SUB = C // SC16
NRING = 3
PH0N = 63
PHASES = ((0, 63), (63, 62))
NP = 10112
RPT = NP // NS
RB = 1000
GRID = N // RB


def _make_edge_pass(feat, gather):
    mesh = plsc.VectorSubcoreMesh(core_axis_name="c", subcore_axis_name="s")
    scratch = [
        pltpu.VMEM((NCH, C), jnp.int32),
        pltpu.VMEM_SHARED((NP, feat), jnp.float32),
        pltpu.SemaphoreType.DMA,
    ]
    if gather:
        scratch.insert(0, pltpu.VMEM((PH0N, C), jnp.int32))
        for _ in range(NRING):
            scratch.append(pltpu.VMEM((C, feat), jnp.float32))
        for _ in range(NRING):
            scratch.append(pltpu.SemaphoreType.DMA)
    else:
        scratch.append(pltpu.VMEM((SC16, feat), jnp.float32))

    if gather:
        def body(hp, srcr, dstr, zer, out, isrc, idst, acc, ssem, *bufs):
            rows = bufs[:NRING]
            gsem = bufs[NRING:]
            c = lax.axis_index("c")
            s = lax.axis_index("s")
            wid = c * NS + s
            rbase = s * RPT
            pltpu.sync_copy(zer.at[pl.ds(rbase, RPT)],
                            acc.at[pl.ds(rbase, RPT)])
            pltpu.sync_copy(dstr.at[wid], idst)
            plsc.subcore_barrier()

            def gath(j, b):
                return pltpu.make_async_copy(hp.at[isrc.at[j]],
                                             rows[b], gsem[b])

            def scat(j, buf):
                for u in range(SUB):
                    pltpu.async_copy(
                        buf.at[pl.ds(u * SC16, SC16)],
                        acc.at[idst.at[j, pl.ds(u * SC16, SC16)]],
                        ssem, add=True)
                for u in range(SUB):
                    pltpu.make_async_copy(
                        buf.at[pl.ds(u * SC16, SC16)],
                        acc.at[idst.at[j, pl.ds(u * SC16, SC16)]],
                        ssem).wait()

            for ph0, phn in PHASES:
                pltpu.sync_copy(srcr.at[wid, pl.ds(ph0, phn)],
                                isrc.at[pl.ds(0, phn)])

                for b in range(NRING):
                    gath(b, b).start()

                def ring(k, carry):
                    for b in range(NRING):
                        jl = k * NRING + b
                        gath(jl, b).wait()
                        scat(ph0 + jl, rows[b])

                        @pl.when(jl + NRING < phn)
                        def _():
                            gath(jl + NRING, b).start()
                    return carry

                lax.fori_loop(0, phn // NRING, ring, 0)
                for r in range(phn % NRING):
                    jl = phn - (phn % NRING) + r
                    gath(jl, jl % NRING).wait()
                    scat(ph0 + jl, rows[jl % NRING])
            plsc.subcore_barrier()
            pltpu.sync_copy(acc.at[pl.ds(rbase, RPT)],
                            out.at[c, pl.ds(rbase, RPT)])
    else:
        def body(ones, dstr, zer, out, idst, acc, ssem, rows0):
            c = lax.axis_index("c")
            s = lax.axis_index("s")
            wid = c * NS + s
            rbase = s * RPT
            pltpu.sync_copy(zer.at[pl.ds(rbase, RPT)],
                            acc.at[pl.ds(rbase, RPT)])
            pltpu.sync_copy(dstr.at[wid], idst)
            pltpu.sync_copy(ones, rows0)
            plsc.subcore_barrier()

            def chunk(j, carry):
                for u in range(SUB):
                    pltpu.async_copy(
                        rows0, acc.at[idst.at[j, pl.ds(u * SC16, SC16)]],
                        ssem, add=True)
                for u in range(SUB):
                    pltpu.make_async_copy(
                        rows0, acc.at[idst.at[j, pl.ds(u * SC16, SC16)]],
                        ssem).wait()
                return carry

            lax.fori_loop(0, NCH, chunk, 0)
            plsc.subcore_barrier()
            pltpu.sync_copy(acc.at[pl.ds(rbase, RPT)],
                            out.at[c, pl.ds(rbase, RPT)])

    return pl.kernel(
        body,
        out_type=jax.ShapeDtypeStruct((NC, NP, feat), jnp.float32),
        mesh=mesh,
        scratch_types=scratch,
        compiler_params=pltpu.CompilerParams(use_tc_tiling_on_sc=False),
    )


_edge128 = _make_edge_pass(D, gather=True)
_edge16 = _make_edge_pass(DP, gather=True)
_deg16 = _make_edge_pass(DP, gather=False)


def _k1_body(x_ref, w_ref, deg_ref, h_ref, dinv_ref):
    deg = deg_ref[0] + deg_ref[1] + 1.0
    dinv = lax.rsqrt(deg)
    dinv_ref[...] = dinv
    h = jnp.dot(x_ref[...], w_ref[...], preferred_element_type=jnp.float32)
    h_ref[...] = h * dinv[:, 0:1]


def _scale_matmul_first(x, w, degp):
    return pl.pallas_call(
        _k1_body,
        grid=(GRID,),
        in_specs=[
            pl.BlockSpec((RB, D), lambda i: (i, 0)),
            pl.BlockSpec((D, D), lambda i: (0, 0)),
            pl.BlockSpec((NC, RB, DP), lambda i: (0, i, 0)),
        ],
        out_specs=[
            pl.BlockSpec((RB, D), lambda i: (i, 0)),
            pl.BlockSpec((RB, DP), lambda i: (i, 0)),
        ],
        out_shape=[
            jax.ShapeDtypeStruct((N, D), jnp.float32),
            jax.ShapeDtypeStruct((N, DP), jnp.float32),
        ],
    )(x, w, degp)


def _k2_body(acc_ref, hprev_ref, dinv_ref, b_ref, w_ref, out_ref):
    dinv = dinv_ref[:, 0:1]
    z = (acc_ref[0] + acc_ref[1] + hprev_ref[...]) * dinv + b_ref[...]
    z = jnp.maximum(z, 0.0)
    out_ref[...] = jnp.dot(z, w_ref[...],
                           preferred_element_type=jnp.float32) * dinv


def _combine_matmul(accp, hprev, dinv16, b2d, w):
    dout = w.shape[1]
    return pl.pallas_call(
        _k2_body,
        grid=(GRID,),
        in_specs=[
            pl.BlockSpec((NC, RB, D), lambda i: (0, i, 0)),
            pl.BlockSpec((RB, D), lambda i: (i, 0)),
            pl.BlockSpec((RB, DP), lambda i: (i, 0)),
            pl.BlockSpec((1, D), lambda i: (0, 0)),
            pl.BlockSpec((D, dout), lambda i: (0, 0)),
        ],
        out_specs=pl.BlockSpec((RB, dout), lambda i: (i, 0)),
        out_shape=jax.ShapeDtypeStruct((N, dout), jnp.float32),
    )(accp, hprev, dinv16, b2d, w)


def _k3_body(acc_ref, h3_ref, dinv_ref, b_ref, batch_ref, out_ref,
             sums_ref, cnt_ref):
    i = pl.program_id(0)

    @pl.when(i == 0)
    def _():
        sums_ref[...] = jnp.zeros_like(sums_ref)
        cnt_ref[...] = jnp.zeros_like(cnt_ref)

    dinv = dinv_ref[:, 0:1]
    z = (acc_ref[0] + acc_ref[1] + h3_ref[...]) * dinv + b_ref[...]
    b = batch_ref[0, 0]
    iota = lax.broadcasted_iota(jnp.int32, (RB, NG), 1)
    onehot = (b[:, None] == iota).astype(jnp.float32)
    sums_ref[...] += lax.dot_general(
        onehot, z, (((0,), (0,)), ((), ())),
        preferred_element_type=jnp.float32)
    cnt_ref[...] += lax.dot_general(
        onehot, jnp.ones((RB, DP), jnp.float32), (((0,), (0,)), ((), ())),
        preferred_element_type=jnp.float32)

    @pl.when(i == GRID - 1)
    def _():
        out_ref[...] = sums_ref[...] / jnp.maximum(cnt_ref[...], 1.0)


def _combine_pool(accp3, h3p, dinv16, b3p2d, batchr):
    return pl.pallas_call(
        _k3_body,
        grid=(GRID,),
        in_specs=[
            pl.BlockSpec((NC, RB, DP), lambda i: (0, i, 0)),
            pl.BlockSpec((RB, DP), lambda i: (i, 0)),
            pl.BlockSpec((RB, DP), lambda i: (i, 0)),
            pl.BlockSpec((1, DP), lambda i: (0, 0)),
            pl.BlockSpec((1, 1, RB), lambda i: (i, 0, 0)),
        ],
        out_specs=pl.BlockSpec((NG, DP), lambda i: (0, 0)),
        out_shape=jax.ShapeDtypeStruct((NG, DP), jnp.float32),
        scratch_shapes=[
            pltpu.VMEM((NG, DP), jnp.float32),
            pltpu.VMEM((NG, DP), jnp.float32),
        ],
        compiler_params=pltpu.CompilerParams(
            dimension_semantics=("arbitrary",)),
    )(accp3, h3p, dinv16, b3p2d, batchr)


def kernel(x, edge_index, batch, W1, b1, W2, b2, W3, b3):
    f32 = jnp.float32
    srcr = edge_index[0].reshape(NW, NCH, C)
    dstr = edge_index[1].reshape(NW, NCH, C)
    zeros128 = jnp.zeros((NP, D), f32)
    zeros16 = jnp.zeros((NP, DP), f32)
    ones16 = jnp.ones((SC16, DP), f32)
    W3p = jnp.pad(W3, ((0, 0), (0, DP - W3.shape[1])))
    b3p = jnp.pad(b3, (0, DP - b3.shape[0])).reshape(1, DP)
    b1_2d = b1.reshape(1, D)
    b2_2d = b2.reshape(1, D)
    batchr = batch.reshape(GRID, 1, RB)

    degp = _deg16(ones16, dstr, zeros16)
    h1p, dinv16 = _scale_matmul_first(x, W1, degp)
    acc1 = _edge128(h1p, srcr, dstr, zeros128)
    h2p = _combine_matmul(acc1, h1p, dinv16, b1_2d, W2)
    acc2 = _edge128(h2p, srcr, dstr, zeros128)
    h3p = _combine_matmul(acc2, h2p, dinv16, b2_2d, W3p)
    acc3 = _edge16(h3p, srcr, dstr, zeros16)
    g16 = _combine_pool(acc3, h3p, dinv16, b3p, batchr)
    return g16[:, :b3.shape[0]]
